# baseline (speedup 1.0000x reference)
import functools

import jax
import jax.numpy as jnp
from jax.experimental import pallas as pl
from jax.experimental.pallas import tpu as pltpu

_Q = 0.5
_BN = 512


def _norms_body(x_ref, n_ref):
    xb = x_ref[0]
    n_ref[0, 0, :] = jnp.sqrt(jnp.sum(xb * xb, axis=-1))


def _mask_body(nsteps, k_lo, k_hi, x_ref, n_ref, o_ref, mask_ref):
    b = pl.program_id(0)
    nb = pl.program_id(1)

    @pl.when((b == 0) & (nb == 0))
    def _compute_mask():
        norms = n_ref[:, 0, :]
        bits = jax.lax.bitcast_convert_type(norms, jnp.int32)
        col = jax.lax.broadcasted_iota(jnp.int32, (1, 2), 1)
        targets = jnp.where(col == 0, k_lo + 1, k_hi + 1)
        B = norms.shape[0]
        lo = jnp.zeros((B, 2), jnp.int32)
        hi = jnp.full((B, 2), 0x7F800000, jnp.int32)

        def body(_, lh):
            lo, hi = lh
            mid = lo + (hi - lo) // 2
            cnt = jnp.sum(
                (bits[:, None, :] <= mid[:, :, None]).astype(jnp.int32),
                axis=-1)
            pred = cnt >= targets
            return jnp.where(pred, lo, mid + 1), jnp.where(pred, mid, hi)

        lo, _ = jax.lax.fori_loop(0, 31, body, (lo, hi))
        v = jax.lax.bitcast_convert_type(lo, jnp.float32)
        thres = v[:, 0:1] * 0.5 + v[:, 1:2] * 0.5
        mask_ref[...] = (~(norms < thres)).astype(jnp.float32)

    m = mask_ref[b, pl.ds(nb * nsteps, nsteps)]
    o_ref[0] = x_ref[0] * m[:, None]


def kernel(x):
    B, N, D = x.shape
    norms = pl.pallas_call(
        _norms_body,
        grid=(B, N // _BN),
        in_specs=[pl.BlockSpec((1, _BN, D), lambda b, nb: (b, nb, 0))],
        out_specs=pl.BlockSpec((1, 1, _BN), lambda b, nb: (b, 0, nb)),
        out_shape=jax.ShapeDtypeStruct((B, 1, N), jnp.float32),
    )(x)

    k_lo = int(_Q * (N - 1))
    k_hi = k_lo + (1 if _Q * (N - 1) != k_lo else 0)
    out = pl.pallas_call(
        functools.partial(_mask_body, _BN, k_lo, k_hi),
        grid=(B, N // _BN),
        in_specs=[
            pl.BlockSpec((1, _BN, D), lambda b, nb: (b, nb, 0)),
            pl.BlockSpec((B, 1, N), lambda b, nb: (0, 0, 0)),
        ],
        out_specs=pl.BlockSpec((1, _BN, D), lambda b, nb: (b, nb, 0)),
        out_shape=jax.ShapeDtypeStruct((B, N, D), x.dtype),
        scratch_shapes=[pltpu.VMEM((B, N), jnp.float32)],
    )(x, norms)
    return out


# speedup vs baseline: 1.5220x; 1.5220x over previous
import functools

import jax
import jax.numpy as jnp
from jax.experimental import pallas as pl
from jax.experimental.pallas import tpu as pltpu

_Q = 0.5
_NC = 8
_SLOTS = 12
_LAG = 2


def _fused_body(k_lo, k_hi, B, N, D, x_hbm, o_hbm, buf, nrm_k, nrm_c,
                sem_in, sem_out):
    R = N // _NC
    S = R // 128
    G = B * _NC

    def in_copy(g):
        b, c = divmod(g, _NC)
        return pltpu.make_async_copy(
            x_hbm.at[b, pl.ds(c * R, R), :],
            buf.at[g % _SLOTS],
            sem_in.at[g % _SLOTS])

    def out_copy(g):
        b, c = divmod(g, _NC)
        return pltpu.make_async_copy(
            buf.at[g % _SLOTS],
            o_hbm.at[b, pl.ds(c * R, R), :],
            sem_out.at[g % _SLOTS])

    col = jax.lax.broadcasted_iota(jnp.int32, (1, 2), 1)
    targets = jnp.where(col == 0, k_lo + 1, k_hi + 1)

    for g in range(_SLOTS):
        in_copy(g).start()

    for b in range(B):
        for c in range(_NC):
            g = b * _NC + c
            in_copy(g).wait()
            xb = buf[g % _SLOTS]
            sq = xb * xb
            nrm_k[pl.ds(c * R, R), :] = jnp.sqrt(
                jnp.sum(sq, axis=1, keepdims=True))
            nrm_c[pl.ds(c * S, S), :] = jnp.sqrt(
                jnp.sum(sq.reshape(S, 128, D), axis=2))

        bits = jax.lax.bitcast_convert_type(nrm_c[...], jnp.int32)

        def srch(_, lh, bits=bits):
            lo, hi = lh
            mid = lo + (hi - lo) // 2
            cnt = jnp.sum((bits[None, None, :, :] <= mid[:, :, None, None])
                          .astype(jnp.int32), axis=(2, 3))
            pred = cnt >= targets
            return jnp.where(pred, lo, mid + 1), jnp.where(pred, mid, hi)

        lo0 = jnp.zeros((1, 2), jnp.int32)
        hi0 = jnp.full((1, 2), 0x7F800000, jnp.int32)
        lo, _ = jax.lax.fori_loop(0, 31, srch, (lo0, hi0))
        v = jax.lax.bitcast_convert_type(lo, jnp.float32)
        thres = v[:, 0:1] * 0.5 + v[:, 1:2] * 0.5

        for c in range(_NC):
            g = b * _NC + c
            m = (~(nrm_k[pl.ds(c * R, R), :] < thres)).astype(jnp.float32)
            buf[g % _SLOTS] = buf[g % _SLOTS] * m
            out_copy(g).start()
            h = g + _SLOTS - _LAG
            if g - _LAG >= 0 and h < G:
                out_copy(g - _LAG).wait()
                in_copy(h).start()

    for g in range(max(G - _SLOTS, 0), G):
        out_copy(g).wait()


def kernel(x):
    B, N, D = x.shape
    k_lo = int(_Q * (N - 1))
    k_hi = k_lo + (1 if _Q * (N - 1) != k_lo else 0)
    R = N // _NC
    return pl.pallas_call(
        functools.partial(_fused_body, k_lo, k_hi, B, N, D),
        in_specs=[pl.BlockSpec(memory_space=pl.ANY)],
        out_specs=pl.BlockSpec(memory_space=pl.ANY),
        out_shape=jax.ShapeDtypeStruct((B, N, D), x.dtype),
        scratch_shapes=[
            pltpu.VMEM((_SLOTS, R, D), jnp.float32),
            pltpu.VMEM((N, 1), jnp.float32),
            pltpu.VMEM((N // 128, 128), jnp.float32),
            pltpu.SemaphoreType.DMA((_SLOTS,)),
            pltpu.SemaphoreType.DMA((_SLOTS,)),
        ],
    )(x)


# speedup vs baseline: 1.5354x; 1.0088x over previous
import functools

import jax
import jax.numpy as jnp
from jax.experimental import pallas as pl
from jax.experimental.pallas import tpu as pltpu

_Q = 0.5
_NC = 16
_SLOTS = 26
_LAG = 4


def _fused_body(k_lo, k_hi, B, N, D, x_hbm, o_hbm, buf, nrm_k, nrm_c,
                sem_in, sem_out):
    R = N // _NC
    S = R // 128
    G = B * _NC

    def in_copy(g):
        b, c = divmod(g, _NC)
        return pltpu.make_async_copy(
            x_hbm.at[b, pl.ds(c * R, R), :],
            buf.at[g % _SLOTS],
            sem_in.at[g % _SLOTS])

    def out_copy(g):
        b, c = divmod(g, _NC)
        return pltpu.make_async_copy(
            buf.at[g % _SLOTS],
            o_hbm.at[b, pl.ds(c * R, R), :],
            sem_out.at[g % _SLOTS])

    col = jax.lax.broadcasted_iota(jnp.int32, (1, 2), 1)
    targets = jnp.where(col == 0, k_lo + 1, k_hi + 1)

    for g in range(_SLOTS):
        in_copy(g).start()

    for b in range(B):
        for c in range(_NC):
            g = b * _NC + c
            in_copy(g).wait()
            xb = buf[g % _SLOTS]
            sq = xb * xb
            nrm_k[pl.ds(c * R, R), :] = jnp.sqrt(
                jnp.sum(sq, axis=1, keepdims=True))
            nrm_c[pl.ds(c * S, S), :] = jnp.sqrt(
                jnp.sum(sq.reshape(S, 128, D), axis=2))

        bits = jax.lax.bitcast_convert_type(nrm_c[...], jnp.int32)

        def srch(_, lh, bits=bits):
            lo, hi = lh
            mid = lo + (hi - lo) // 2
            cnt = jnp.sum((bits[None, None, :, :] <= mid[:, :, None, None])
                          .astype(jnp.int32), axis=(2, 3))
            pred = cnt >= targets
            return jnp.where(pred, lo, mid + 1), jnp.where(pred, mid, hi)

        lo0 = jnp.zeros((1, 2), jnp.int32)
        hi0 = jnp.full((1, 2), 0x7F800000, jnp.int32)
        lo, _ = jax.lax.fori_loop(0, 31, srch, (lo0, hi0))
        v = jax.lax.bitcast_convert_type(lo, jnp.float32)
        thres = v[:, 0:1] * 0.5 + v[:, 1:2] * 0.5

        for c in range(_NC):
            g = b * _NC + c
            m = (~(nrm_k[pl.ds(c * R, R), :] < thres)).astype(jnp.float32)
            buf[g % _SLOTS] = buf[g % _SLOTS] * m
            out_copy(g).start()
            h = g + _SLOTS - _LAG
            if g - _LAG >= 0 and h < G:
                out_copy(g - _LAG).wait()
                in_copy(h).start()

    for g in range(max(G - _SLOTS, 0), G):
        out_copy(g).wait()


def kernel(x):
    B, N, D = x.shape
    k_lo = int(_Q * (N - 1))
    k_hi = k_lo + (1 if _Q * (N - 1) != k_lo else 0)
    R = N // _NC
    return pl.pallas_call(
        functools.partial(_fused_body, k_lo, k_hi, B, N, D),
        in_specs=[pl.BlockSpec(memory_space=pl.ANY)],
        out_specs=pl.BlockSpec(memory_space=pl.ANY),
        out_shape=jax.ShapeDtypeStruct((B, N, D), x.dtype),
        scratch_shapes=[
            pltpu.VMEM((_SLOTS, R, D), jnp.float32),
            pltpu.VMEM((N, 1), jnp.float32),
            pltpu.VMEM((N // 128, 128), jnp.float32),
            pltpu.SemaphoreType.DMA((_SLOTS,)),
            pltpu.SemaphoreType.DMA((_SLOTS,)),
        ],
    )(x)
